# baseline (speedup 1.0000x reference)
import functools
import jax
import jax.numpy as jnp
from jax import lax
from jax.experimental import pallas as pl
from jax.experimental.pallas import tpu as pltpu
from jax.experimental.pallas import tpu_sc as plsc

NC = 2
NS = 16
NBUF = 4
PRIME = 2


def _sc_broadcast(W_pos, nbatch, seq_len, d_model):
    nw = NC * NS
    rows_per_w = seq_len // nw
    R = 8
    nchunks = rows_per_w // R
    mesh = plsc.VectorSubcoreMesh(core_axis_name="c", subcore_axis_name="s")

    @functools.partial(
        pl.kernel,
        mesh=mesh,
        out_type=jax.ShapeDtypeStruct((nbatch * seq_len, d_model), W_pos.dtype),
        scratch_types=[
            pltpu.VMEM((NBUF, R, d_model), jnp.float32),
            pltpu.SemaphoreType.DMA((NBUF,)),
            pltpu.SemaphoreType.DMA((NBUF,)),
        ],
    )
    def k(w_hbm, out_hbm, buf, sem_in, sem_out):
        wid = lax.axis_index("s") * NC + lax.axis_index("c")
        base = wid * rows_per_w

        def start_in(c):
            cp = pltpu.make_async_copy(
                w_hbm.at[pl.ds(base + c * R, R)], buf.at[c % NBUF], sem_in.at[c % NBUF]
            )
            cp.start()
            return cp

        def start_outs(c):
            cps = []
            for b in range(nbatch):
                cp = pltpu.make_async_copy(
                    buf.at[c % NBUF],
                    out_hbm.at[pl.ds(b * seq_len + base + c * R, R)],
                    sem_out.at[c % NBUF],
                )
                cp.start()
                cps.append(cp)
            return cps

        in_cp = {}
        outs = {}
        drained = set()
        for c in range(min(PRIME, nchunks)):
            in_cp[c] = start_in(c)
        for c in range(nchunks):
            in_cp[c].wait()
            outs[c] = start_outs(c)
            nxt = c + PRIME
            if nxt < nchunks:
                prev = nxt - NBUF
                if prev >= 0:
                    for w in outs[prev]:
                        w.wait()
                    drained.add(prev)
                in_cp[nxt] = start_in(nxt)
        for c in range(nchunks):
            if c not in drained:
                for w in outs[c]:
                    w.wait()

    return k(W_pos)


def _make_tc_body(nbatch, bs, d_model):
    def _body(w_ref, o_ref):
        o_ref[...] = jnp.broadcast_to(w_ref[...][None], (nbatch, bs, d_model))
    return _body


def _tc_broadcast(W_pos, nbatch, seq_len, d_model):
    bs = 512
    grid = (seq_len // bs,)
    return pl.pallas_call(
        _make_tc_body(nbatch, bs, d_model),
        grid=grid,
        in_specs=[pl.BlockSpec((bs, d_model), lambda s: (s, 0))],
        out_specs=pl.BlockSpec((nbatch, bs, d_model), lambda s: (0, s, 0)),
        out_shape=jax.ShapeDtypeStruct((nbatch, seq_len, d_model), W_pos.dtype),
    )(W_pos)


def kernel(tokens, W_pos):
    batch, seq_len = tokens.shape
    d_model = W_pos.shape[1]
    n_tc = batch - 1
    out_tc = _tc_broadcast(W_pos, n_tc, seq_len, d_model)
    out_sc = _sc_broadcast(W_pos, 1, seq_len, d_model)
    return jnp.concatenate([out_tc, out_sc.reshape(1, seq_len, d_model)], axis=0)


# speedup vs baseline: 3.6049x; 3.6049x over previous
import jax
import jax.numpy as jnp
from jax.experimental import pallas as pl
from jax.experimental.pallas import tpu as pltpu

NBUF = 3
PRIME = 2


def _make_body(batch, seq_len, bs):
    nchunks = seq_len // bs

    def _body(w_hbm, o_hbm, buf, sem_in, sem_out):
        def start_in(c):
            cp = pltpu.make_async_copy(
                w_hbm.at[pl.ds(c * bs, bs)], buf.at[c % NBUF], sem_in.at[c % NBUF]
            )
            cp.start()
            return cp

        def start_outs(c):
            cps = []
            for b in range(batch):
                cp = pltpu.make_async_copy(
                    buf.at[c % NBUF],
                    o_hbm.at[b, pl.ds(c * bs, bs)],
                    sem_out.at[c % NBUF],
                )
                cp.start()
                cps.append(cp)
            return cps

        in_cp = {}
        outs = {}
        drained = set()
        for c in range(min(PRIME, nchunks)):
            in_cp[c] = start_in(c)
        for c in range(nchunks):
            in_cp[c].wait()
            outs[c] = start_outs(c)
            nxt = c + PRIME
            if nxt < nchunks:
                prev = nxt - NBUF
                if prev >= 0:
                    for w in outs[prev]:
                        w.wait()
                    drained.add(prev)
                in_cp[nxt] = start_in(nxt)
        for c in range(nchunks):
            if c not in drained:
                for w in outs[c]:
                    w.wait()

    return _body


def kernel(tokens, W_pos):
    batch, seq_len = tokens.shape
    d_model = W_pos.shape[1]
    bs = 512
    return pl.pallas_call(
        _make_body(batch, seq_len, bs),
        in_specs=[pl.BlockSpec(memory_space=pl.ANY)],
        out_specs=pl.BlockSpec(memory_space=pl.ANY),
        out_shape=jax.ShapeDtypeStruct((batch, seq_len, d_model), W_pos.dtype),
        scratch_shapes=[
            pltpu.VMEM((NBUF, bs, d_model), jnp.float32),
            pltpu.SemaphoreType.DMA((NBUF,)),
            pltpu.SemaphoreType.DMA((NBUF,)),
        ],
    )(W_pos)
